# baseline (speedup 1.0000x reference)
import functools

import jax
import jax.numpy as jnp
from jax import lax
from jax.experimental import pallas as pl
from jax.experimental.pallas import tpu as pltpu, tpu_sc as plsc

D_MODEL = 768
D_VOCAB = 100000

_B = 32768
_CHUNK = 16
_NBUF = 8
_LOOK = 6


def _make_gather():
    info = plsc.get_sparse_core_info()
    nc, ns = info.num_cores, info.num_subcores
    nw = nc * ns
    b_per_w = _B // nw
    n_steps = b_per_w // _CHUNK
    mesh = plsc.VectorSubcoreMesh(core_axis_name="c", subcore_axis_name="s")

    @functools.partial(
        pl.kernel,
        mesh=mesh,
        out_type=jax.ShapeDtypeStruct((_B, D_MODEL), jnp.float32),
        scratch_types=[
            pltpu.VMEM((b_per_w,), jnp.int32),
            pltpu.VMEM((_NBUF, _CHUNK, D_MODEL), jnp.float32),
        ] + [pltpu.SemaphoreType.DMA] * (2 * _NBUF),
    )
    def gather_k(table_hbm, idx_hbm, out_hbm, idx_v, rows_v, *sems):
        gsems = sems[:_NBUF]
        wsems = sems[_NBUF:]
        wid = lax.axis_index("s") * nc + lax.axis_index("c")
        base = wid * b_per_w

        pltpu.sync_copy(idx_hbm.at[pl.ds(base, b_per_w)], idx_v)

        def start_gather(i, b):
            pltpu.async_copy(
                table_hbm.at[idx_v.at[pl.ds(i * _CHUNK, _CHUNK)]],
                rows_v.at[b], gsems[b])

        def wait_gather(i, b):
            pltpu.make_async_copy(
                table_hbm.at[idx_v.at[pl.ds(i * _CHUNK, _CHUNK)]],
                rows_v.at[b], gsems[b]).wait()

        def start_write(i, b):
            pltpu.async_copy(
                rows_v.at[b],
                out_hbm.at[pl.ds(base + i * _CHUNK, _CHUNK)], wsems[b])

        def wait_write(i, b):
            pltpu.make_async_copy(
                rows_v.at[b],
                out_hbm.at[pl.ds(base + i * _CHUNK, _CHUNK)], wsems[b]).wait()

        for u in range(_LOOK):
            start_gather(u, u)

        def ring_body(p, carry):
            for u in range(_NBUF):
                j = p * _NBUF + u
                g = j + _LOOK
                bg = (u + _LOOK) % _NBUF

                @pl.when(g < n_steps)
                def _():
                    @pl.when(g >= _NBUF)
                    def _():
                        wait_write(g - _NBUF, bg)
                    start_gather(g, bg)

                wait_gather(j, u)
                start_write(j, u)
            return carry

        lax.fori_loop(0, n_steps // _NBUF, ring_body, 0)
        for u in range(_NBUF):
            wait_write(n_steps - _NBUF + u, (n_steps - _NBUF + u) % _NBUF)

    return gather_k


def kernel(x, W_E):
    xf = x.reshape(_B).astype(jnp.int32)
    out = _make_gather()(W_E.T, xf)
    return out.reshape(x.shape[0], x.shape[1], D_MODEL)


# speedup vs baseline: 1.0125x; 1.0125x over previous
import functools

import jax
import jax.numpy as jnp
from jax import lax
from jax.experimental import pallas as pl
from jax.experimental.pallas import tpu as pltpu, tpu_sc as plsc

D_MODEL = 768
D_VOCAB = 100000

_B = 32768
_CHUNK = 16
_NBUF = 8
_LOOK = 4


def _make_gather():
    info = plsc.get_sparse_core_info()
    nc, ns = info.num_cores, info.num_subcores
    nw = nc * ns
    b_per_w = _B // nw
    n_steps = b_per_w // _CHUNK
    mesh = plsc.VectorSubcoreMesh(core_axis_name="c", subcore_axis_name="s")

    @functools.partial(
        pl.kernel,
        mesh=mesh,
        out_type=jax.ShapeDtypeStruct((_B, D_MODEL), jnp.float32),
        scratch_types=[
            pltpu.VMEM((b_per_w,), jnp.int32),
            pltpu.VMEM((_NBUF, _CHUNK, D_MODEL), jnp.float32),
        ] + [pltpu.SemaphoreType.DMA] * (2 * _NBUF),
    )
    def gather_k(table_hbm, idx_hbm, out_hbm, idx_v, rows_v, *sems):
        gsems = sems[:_NBUF]
        wsems = sems[_NBUF:]
        wid = lax.axis_index("s") * nc + lax.axis_index("c")
        base = wid * b_per_w

        pltpu.sync_copy(idx_hbm.at[pl.ds(base, b_per_w)], idx_v)

        def start_gather(i, b):
            pltpu.async_copy(
                table_hbm.at[idx_v.at[pl.ds(i * _CHUNK, _CHUNK)]],
                rows_v.at[b], gsems[b])

        def wait_gather(i, b):
            pltpu.make_async_copy(
                table_hbm.at[idx_v.at[pl.ds(i * _CHUNK, _CHUNK)]],
                rows_v.at[b], gsems[b]).wait()

        def start_write(i, b):
            pltpu.async_copy(
                rows_v.at[b],
                out_hbm.at[pl.ds(base + i * _CHUNK, _CHUNK)], wsems[b])

        def wait_write(i, b):
            pltpu.make_async_copy(
                rows_v.at[b],
                out_hbm.at[pl.ds(base + i * _CHUNK, _CHUNK)], wsems[b]).wait()

        for u in range(_LOOK):
            start_gather(u, u)

        def ring_body(p, carry):
            for u in range(_NBUF):
                j = p * _NBUF + u
                g = j + _LOOK
                bg = (u + _LOOK) % _NBUF

                @pl.when(g < n_steps)
                def _():
                    @pl.when(g >= _NBUF)
                    def _():
                        wait_write(g - _NBUF, bg)
                    start_gather(g, bg)

                wait_gather(j, u)
                start_write(j, u)
            return carry

        lax.fori_loop(0, n_steps // _NBUF, ring_body, 0)
        for u in range(_NBUF):
            wait_write(n_steps - _NBUF + u, (n_steps - _NBUF + u) % _NBUF)

    return gather_k


def kernel(x, W_E):
    xf = x.reshape(_B).astype(jnp.int32)
    out = _make_gather()(W_E.T, xf)
    return out.reshape(x.shape[0], x.shape[1], D_MODEL)
